# Initial kernel scaffold; baseline (speedup 1.0000x reference)
import jax
import jax.numpy as jnp
from jax.experimental import pallas as pl


def kernel(x, means, alpha):
    raise NotImplementedError("write your pallas kernel here")


# speedup vs baseline: 2.6679x; 2.6679x over previous
import functools

import jax
import jax.numpy as jnp
from jax.experimental import pallas as pl
from jax.experimental.pallas import tpu as pltpu

_LANE = 128


def _round_up(v, m):
    return (v + m - 1) // m * m


def _maha_kernel(x_ref, b_ref, t4_ref, out_ref, *, c_pad):
    x = x_ref[...]
    xb = x.astype(jnp.bfloat16)
    r = jnp.dot(xb, b_ref[...], preferred_element_type=jnp.float32)
    term23 = r[:, :c_pad]
    xa = r[:, c_pad:]
    term1 = jnp.sum(xa * x, axis=1, keepdims=True)
    qmin = term1 + jnp.min(t4_ref[...] - term23, axis=1, keepdims=True)
    packed = jnp.broadcast_to(qmin, (qmin.shape[0], _LANE))
    out_ref[...] = packed.T[:1, :]


def kernel(x, means, alpha):
    n, d = x.shape
    d_m, c = means.shape
    assert d == d_m and alpha.shape == (d, d)

    f32 = jnp.float32
    x = x.astype(f32)
    means = means.astype(f32)
    alpha = alpha.astype(f32)

    hi = jax.lax.Precision.HIGHEST
    am = jnp.dot(alpha, means, precision=hi)
    m2 = jnp.dot(alpha + alpha.T, means, precision=hi)
    t4 = jnp.sum(means * am, axis=0, keepdims=True)

    c_pad = _round_up(c, _LANE)
    b_mat = jnp.zeros((d, c_pad + d), f32)
    b_mat = b_mat.at[:, :c].set(m2)
    b_mat = b_mat.at[:, c_pad:].set(alpha)
    b_bf = b_mat.astype(jnp.bfloat16)
    t4_p = jnp.full((1, c_pad), jnp.inf, f32).at[:, :c].set(t4)

    tn = min(2048, _round_up(n, _LANE))
    n_pad = _round_up(n, tn)
    num_tiles = n_pad // tn
    x_p = x if n_pad == n else jnp.zeros((n_pad, d), f32).at[:n, :].set(x)

    out = pl.pallas_call(
        functools.partial(_maha_kernel, c_pad=c_pad),
        out_shape=jax.ShapeDtypeStruct((num_tiles, 1, tn), f32),
        grid=(num_tiles,),
        in_specs=[
            pl.BlockSpec((tn, d), lambda i: (i, 0)),
            pl.BlockSpec((d, c_pad + d), lambda i: (0, 0),
                         pipeline_mode=pl.Buffered(1)),
            pl.BlockSpec((1, c_pad), lambda i: (0, 0),
                         pipeline_mode=pl.Buffered(1)),
        ],
        out_specs=pl.BlockSpec((None, 1, tn), lambda i: (i, 0, 0)),
        compiler_params=pltpu.CompilerParams(
            dimension_semantics=("parallel",),
            vmem_limit_bytes=48 << 20,
        ),
    )(x_p, b_bf, t4_p)

    return out.reshape(n_pad)[:n]


# speedup vs baseline: 3.1338x; 1.1746x over previous
import functools

import jax
import jax.numpy as jnp
from jax.experimental import pallas as pl
from jax.experimental.pallas import tpu as pltpu

_LANE = 128


def _round_up(v, m):
    return (v + m - 1) // m * m


def _maha_kernel(x_ref, b_ref, t4_ref, out_ref, *, c_pad):
    x = x_ref[...]
    xb = x.astype(jnp.bfloat16)
    r = jnp.dot(xb, b_ref[...], preferred_element_type=jnp.float32)
    term23 = r[:, :c_pad]
    xa = r[:, c_pad:]
    term1 = jnp.sum(xa * x, axis=1, keepdims=True)
    qmin = term1 + jnp.min(t4_ref[...] - term23, axis=1, keepdims=True)
    packed = jnp.broadcast_to(qmin, (qmin.shape[0], _LANE))
    out_ref[...] = packed.T[:1, :]


def kernel(x, means, alpha):
    n, d = x.shape
    d_m, c = means.shape
    assert d == d_m and alpha.shape == (d, d)

    f32 = jnp.float32
    x = x.astype(f32)
    means = means.astype(f32)
    alpha = alpha.astype(f32)

    hi = jax.lax.Precision.HIGHEST
    am = jnp.dot(alpha, means, precision=hi)
    m2 = jnp.dot(alpha + alpha.T, means, precision=hi)
    t4 = jnp.sum(means * am, axis=0, keepdims=True)

    c_pad = _round_up(c, _LANE)
    b_mat = jnp.zeros((d, c_pad + d), f32)
    b_mat = b_mat.at[:, :c].set(m2)
    b_mat = b_mat.at[:, c_pad:].set(alpha)
    b_bf = b_mat.astype(jnp.bfloat16)
    t4_p = jnp.full((1, c_pad), jnp.inf, f32).at[:, :c].set(t4)

    tn = min(4096, _round_up(n, _LANE))
    n_pad = _round_up(n, tn)
    num_tiles = n_pad // tn
    x_p = x if n_pad == n else jnp.zeros((n_pad, d), f32).at[:n, :].set(x)

    out = pl.pallas_call(
        functools.partial(_maha_kernel, c_pad=c_pad),
        out_shape=jax.ShapeDtypeStruct((num_tiles, 1, tn), f32),
        grid=(num_tiles,),
        in_specs=[
            pl.BlockSpec((tn, d), lambda i: (i, 0)),
            pl.BlockSpec((d, c_pad + d), lambda i: (0, 0),
                         pipeline_mode=pl.Buffered(1)),
            pl.BlockSpec((1, c_pad), lambda i: (0, 0),
                         pipeline_mode=pl.Buffered(1)),
        ],
        out_specs=pl.BlockSpec((None, 1, tn), lambda i: (i, 0, 0)),
        compiler_params=pltpu.CompilerParams(
            dimension_semantics=("parallel",),
            vmem_limit_bytes=48 << 20,
        ),
    )(x_p, b_bf, t4_p)

    return out.reshape(n_pad)[:n]


# speedup vs baseline: 3.2307x; 1.0309x over previous
import functools

import jax
import jax.numpy as jnp
from jax.experimental import pallas as pl
from jax.experimental.pallas import tpu as pltpu

_LANE = 128


def _round_up(v, m):
    return (v + m - 1) // m * m


def _maha_kernel(x_ref, b_ref, t4_ref, out_ref, *, c_pad):
    x = x_ref[...]
    xb = x.astype(jnp.bfloat16)
    r = jnp.dot(xb, b_ref[...], preferred_element_type=jnp.float32)
    term23 = r[:, :c_pad]
    xa = r[:, c_pad:]
    term1 = jnp.sum(xa * x, axis=1, keepdims=True)
    qmin = term1 + jnp.min(t4_ref[...] - term23, axis=1, keepdims=True)
    packed = jnp.broadcast_to(qmin, (qmin.shape[0], _LANE))
    out_ref[...] = packed.T[:1, :]


def kernel(x, means, alpha):
    n, d = x.shape
    d_m, c = means.shape
    assert d == d_m and alpha.shape == (d, d)

    f32 = jnp.float32
    x = x.astype(f32)
    means = means.astype(f32)
    alpha = alpha.astype(f32)

    hi = jax.lax.Precision.HIGHEST
    am = jnp.dot(alpha, means, precision=hi)
    m2 = jnp.dot(alpha + alpha.T, means, precision=hi)
    t4 = jnp.sum(means * am, axis=0, keepdims=True)

    c_pad = _round_up(c, _LANE)
    b_mat = jnp.zeros((d, c_pad + d), f32)
    b_mat = b_mat.at[:, :c].set(m2)
    b_mat = b_mat.at[:, c_pad:].set(alpha)
    b_bf = b_mat.astype(jnp.bfloat16)
    t4_p = jnp.full((1, c_pad), jnp.inf, f32).at[:, :c].set(t4)

    tn = min(8192, _round_up(n, _LANE))
    n_pad = _round_up(n, tn)
    num_tiles = n_pad // tn
    x_p = x if n_pad == n else jnp.zeros((n_pad, d), f32).at[:n, :].set(x)

    out = pl.pallas_call(
        functools.partial(_maha_kernel, c_pad=c_pad),
        out_shape=jax.ShapeDtypeStruct((num_tiles, 1, tn), f32),
        grid=(num_tiles,),
        in_specs=[
            pl.BlockSpec((tn, d), lambda i: (i, 0)),
            pl.BlockSpec((d, c_pad + d), lambda i: (0, 0),
                         pipeline_mode=pl.Buffered(1)),
            pl.BlockSpec((1, c_pad), lambda i: (0, 0),
                         pipeline_mode=pl.Buffered(1)),
        ],
        out_specs=pl.BlockSpec((None, 1, tn), lambda i: (i, 0, 0)),
        compiler_params=pltpu.CompilerParams(
            dimension_semantics=("parallel",),
            vmem_limit_bytes=48 << 20,
        ),
    )(x_p, b_bf, t4_p)

    return out.reshape(n_pad)[:n]


# speedup vs baseline: 3.7336x; 1.1557x over previous
import jax
import jax.numpy as jnp
from jax.experimental import pallas as pl
from jax.experimental.pallas import tpu as pltpu

_LANE = 128
_TN = 8192


def _round_up(v, m):
    return (v + m - 1) // m * m


def _maha_kernel(x_ref, means_ref, alpha_ref, out_ref):
    bf16 = jnp.bfloat16
    f32 = jnp.float32
    mb = means_ref[...]
    abb = alpha_ref[...].astype(bf16)
    mbb = mb.astype(bf16)
    am = jnp.dot(abb, mbb, preferred_element_type=f32)
    atm = jax.lax.dot_general(abb, mbb, (((0,), (0,)), ((), ())),
                              preferred_element_type=f32)
    t4 = jnp.sum(mb * am, axis=0, keepdims=True)
    m2 = (am + atm).astype(bf16)

    x = x_ref[...]
    xb = x.astype(bf16)
    term23 = jnp.dot(xb, m2, preferred_element_type=f32)
    xa = jnp.dot(xb, abb, preferred_element_type=f32)
    term1 = jnp.sum(xa * x, axis=1, keepdims=True)
    qmin = term1 + jnp.min(t4 - term23, axis=1, keepdims=True)
    packed = jnp.broadcast_to(qmin, (qmin.shape[0], _LANE))
    out_ref[...] = packed.T[:1, :]


def kernel(x, means, alpha):
    n, d = x.shape
    d_m, c = means.shape
    assert d == d_m and alpha.shape == (d, d)

    f32 = jnp.float32
    x = x.astype(f32)
    means = means.astype(f32)
    alpha = alpha.astype(f32)

    tn = min(_TN, _round_up(n, _LANE))
    n_pad = _round_up(n, tn)
    num_tiles = n_pad // tn
    x_p = x if n_pad == n else jnp.zeros((n_pad, d), f32).at[:n, :].set(x)

    out = pl.pallas_call(
        _maha_kernel,
        out_shape=jax.ShapeDtypeStruct((num_tiles, 1, tn), f32),
        grid=(num_tiles,),
        in_specs=[
            pl.BlockSpec((tn, d), lambda i: (i, 0)),
            pl.BlockSpec((d, c), lambda i: (0, 0),
                         pipeline_mode=pl.Buffered(1)),
            pl.BlockSpec((d, d), lambda i: (0, 0),
                         pipeline_mode=pl.Buffered(1)),
        ],
        out_specs=pl.BlockSpec((None, 1, tn), lambda i: (i, 0, 0)),
        compiler_params=pltpu.CompilerParams(
            dimension_semantics=("parallel",),
            vmem_limit_bytes=56 << 20,
        ),
    )(x_p, means, alpha)

    return out.reshape(n_pad)[:n]
